# baseline (speedup 1.0000x reference)
import functools

import jax
import jax.numpy as jnp
from jax.experimental import pallas as pl
from jax.experimental.pallas import tpu as pltpu


def _mm_relu_kernel(x_ref, w_ref, b_ref, o_ref, *, relu):
    acc = jnp.dot(x_ref[...], w_ref[...], preferred_element_type=jnp.float32)
    acc = acc + b_ref[...]
    if relu:
        acc = jnp.maximum(acc, 0.0)
    o_ref[...] = acc


def _mm(x, w, b, relu=True):
    m, k = x.shape
    n = w.shape[1]
    return pl.pallas_call(
        functools.partial(_mm_relu_kernel, relu=relu),
        out_shape=jax.ShapeDtypeStruct((m, n), jnp.float32),
    )(x, w, b[None, :])


def _gat(x, src, dst, W, a_s, a_d, b, heads, out_ch):
    n = x.shape[0]
    loop = jnp.arange(n, dtype=src.dtype)
    s = jnp.concatenate([src, loop])
    d = jnp.concatenate([dst, loop])
    h = (x @ W).reshape(n, heads, out_ch)
    al_s = jnp.sum(h * a_s, axis=-1)
    al_d = jnp.sum(h * a_d, axis=-1)
    e = al_s[s] + al_d[d]
    e = jnp.where(e > 0, e, 0.2 * e)
    m = jax.ops.segment_max(e, d, num_segments=n)
    ex = jnp.exp(e - m[d])
    z = jax.ops.segment_sum(ex, d, num_segments=n)
    alpha = ex / (z[d] + 1e-16)
    agg = jax.ops.segment_sum(h[s] * alpha[:, :, None], d, num_segments=n)
    return agg.reshape(n, heads * out_ch) + b


def kernel(x, edge_index, batch, data_descriptor_ECFP, W1, a1s, a1d, b1, W2, a2s, a2d, b2, W3, a3s, a3d, b3, W4, a4s, a4d, b4, Wg, bg, Wf1, bf1, Wf2, bf2, Wf3, bf3, Wf4, bf4, Wf5, bf5):
    src, dst = edge_index[0], edge_index[1]
    h = jax.nn.relu(_gat(x, src, dst, W1, a1s, a1d, b1, 5, 114))
    h = jax.nn.relu(_gat(h, src, dst, W2, a2s, a2d, b2, 5, 171))
    h = jax.nn.relu(_gat(h, src, dst, W3, a3s, a3d, b3, 5, 114))
    h = jax.nn.relu(_gat(h, src, dst, W4, a4s, a4d, b4, 1, 570))
    g = jax.ops.segment_sum(h, batch, num_segments=1024)
    g = g @ Wg + bg
    c = jnp.concatenate([g, data_descriptor_ECFP], axis=1)
    o = _mm(c, Wf1, bf1)
    o = _mm(o, Wf2, bf2)
    o = _mm(o, Wf3, bf3)
    o = _mm(o, Wf4, bf4)
    return _mm(o, Wf5, bf5, relu=False)
